# Initial kernel scaffold; baseline (speedup 1.0000x reference)
import functools

import jax
import jax.numpy as jnp
from jax import lax
from jax.experimental import pallas as pl
from jax.experimental.pallas import tpu as pltpu
from jax.experimental.pallas import tpu_sc as plsc

N, E, D, R, G = 10000, 320000, 128, 4, 64
HID, F2, OUT = 32, 64, 10

NC, NS = 2, 16
NW = NC * NS
CH = 128
EPT = E // NW
NCHUNK = -(-EPT // CH)
EPT_PAD = NCHUNK * CH
E_PAD = EPT_PAD * NW
NPAD = 10112
RPT = NPAD // NS
DEGW = 16
NB = 1000
NBLK = N // NB


def _edge_pass_body(with_deg, F, table_hbm, gidx_hbm, didx_hbm, *refs):
    if with_deg:
        acc_out, deg_out, gidx_v, didx_v, rows_v, acc_sh, ones_v, deg_sh = refs
    else:
        acc_out, gidx_v, didx_v, rows_v, acc_sh = refs
    c = lax.axis_index("c")
    s = lax.axis_index("s")
    wid = s * NC + c

    pltpu.sync_copy(gidx_hbm.at[wid], gidx_v)
    pltpu.sync_copy(didx_hbm.at[wid], didx_v)

    zv = jnp.zeros((16,), jnp.float32)

    def zrow(i, _):
        for k in range(F // 16):
            rows_v[i, pl.ds(k * 16, 16)] = zv
        return 0

    lax.fori_loop(0, CH, zrow, 0)
    base = s * RPT
    for off in range(0, RPT, CH):
        ln = min(CH, RPT - off)
        pltpu.sync_copy(rows_v.at[pl.ds(0, ln)], acc_sh.at[pl.ds(base + off, ln)])

    if with_deg:
        def zrow2(i, _):
            ones_v[i, pl.ds(0, 16)] = zv
            return 0

        lax.fori_loop(0, CH, zrow2, 0)
        for off in range(0, RPT, CH):
            ln = min(CH, RPT - off)
            pltpu.sync_copy(ones_v.at[pl.ds(0, ln)], deg_sh.at[pl.ds(base + off, ln)])
        ov = jnp.ones((16,), jnp.float32)

        def orow(i, _):
            ones_v[i, pl.ds(0, 16)] = ov
            return 0

        lax.fori_loop(0, CH, orow, 0)

    plsc.subcore_barrier()

    def chunk(j, _):
        pltpu.sync_copy(table_hbm.at[gidx_v.at[j]], rows_v)
        pltpu.sync_copy(rows_v, acc_sh.at[didx_v.at[j]], add=True)
        if with_deg:
            pltpu.sync_copy(ones_v, deg_sh.at[didx_v.at[j]], add=True)
        return 0

    lax.fori_loop(0, NCHUNK, chunk, 0)

    plsc.subcore_barrier()

    pltpu.sync_copy(acc_sh.at[pl.ds(base, RPT)], acc_out.at[c, pl.ds(base, RPT)])
    if with_deg:
        pltpu.sync_copy(deg_sh.at[pl.ds(base, RPT)], deg_out.at[c, pl.ds(base, RPT)])


def _make_edge_pass(F, with_deg):
    mesh = plsc.VectorSubcoreMesh(core_axis_name="c", subcore_axis_name="s")
    out_type = [jax.ShapeDtypeStruct((NC, NPAD, F), jnp.float32)]
    scratch = [
        pltpu.VMEM((NCHUNK, CH), jnp.int32),
        pltpu.VMEM((NCHUNK, CH), jnp.int32),
        pltpu.VMEM((CH, F), jnp.float32),
        pltpu.VMEM_SHARED((NPAD, F), jnp.float32),
    ]
    if with_deg:
        out_type.append(jax.ShapeDtypeStruct((NC, NPAD, DEGW), jnp.float32))
        scratch += [
            pltpu.VMEM((CH, DEGW), jnp.float32),
            pltpu.VMEM_SHARED((NPAD, DEGW), jnp.float32),
        ]
    return pl.kernel(
        functools.partial(_edge_pass_body, with_deg, F),
        out_type=out_type,
        mesh=mesh,
        scratch_types=scratch,
    )


_edge_pass_rgcn = _make_edge_pass(HID, True)
_edge_pass_gcn = _make_edge_pass(F2, False)


def _tc_a_body(x_ref, wrel_ref, wself_ref, brg_ref, hrel_ref, selfp_ref):
    xb = x_ref[...]
    for r in range(R):
        hrel_ref[r] = jnp.dot(xb, wrel_ref[r], preferred_element_type=jnp.float32)
    selfp_ref[...] = (
        jnp.dot(xb, wself_ref[...], preferred_element_type=jnp.float32)
        + brg_ref[...]
    )


_tc_a = pl.pallas_call(
    _tc_a_body,
    grid=(NBLK,),
    in_specs=[
        pl.BlockSpec((NB, D), lambda i: (i, 0)),
        pl.BlockSpec((R, D, HID), lambda i: (0, 0, 0)),
        pl.BlockSpec((D, HID), lambda i: (0, 0)),
        pl.BlockSpec((1, HID), lambda i: (0, 0)),
    ],
    out_specs=[
        pl.BlockSpec((R, NB, HID), lambda i: (0, i, 0)),
        pl.BlockSpec((NB, HID), lambda i: (i, 0)),
    ],
    out_shape=[
        jax.ShapeDtypeStruct((R, N, HID), jnp.float32),
        jax.ShapeDtypeStruct((N, HID), jnp.float32),
    ],
)


def _tc_c_body(acc_ref, selfp_ref, degp_ref, w1_ref, g1_ref, dinv_ref):
    h0 = jnp.maximum(acc_ref[0] + acc_ref[1] + selfp_ref[...], 0.0)
    deg = degp_ref[0] + degp_ref[1] + 1.0
    dinv = lax.rsqrt(deg)
    dinv_ref[...] = dinv
    g1_ref[...] = dinv[:, 0:1] * jnp.dot(
        h0, w1_ref[...], preferred_element_type=jnp.float32)


_tc_c = pl.pallas_call(
    _tc_c_body,
    grid=(NBLK,),
    in_specs=[
        pl.BlockSpec((NC, NB, HID), lambda i: (0, i, 0)),
        pl.BlockSpec((NB, HID), lambda i: (i, 0)),
        pl.BlockSpec((NC, NB, DEGW), lambda i: (0, i, 0)),
        pl.BlockSpec((HID, F2), lambda i: (0, 0)),
    ],
    out_specs=[
        pl.BlockSpec((NB, F2), lambda i: (i, 0)),
        pl.BlockSpec((NB, DEGW), lambda i: (i, 0)),
    ],
    out_shape=[
        jax.ShapeDtypeStruct((N, F2), jnp.float32),
        jax.ShapeDtypeStruct((N, DEGW), jnp.float32),
    ],
)


def _tc_e_body(acc_ref, g1_ref, dinv_ref, w3_ref, b1_ref, g2_ref):
    dinv1 = dinv_ref[:, 0:1]
    out1 = jnp.maximum(
        dinv1 * (acc_ref[0] + acc_ref[1] + g1_ref[...]) + b1_ref[...], 0.0)
    g2_ref[...] = dinv1 * jnp.dot(
        out1, w3_ref[...], preferred_element_type=jnp.float32)


_tc_e = pl.pallas_call(
    _tc_e_body,
    grid=(NBLK,),
    in_specs=[
        pl.BlockSpec((NC, NB, F2), lambda i: (0, i, 0)),
        pl.BlockSpec((NB, F2), lambda i: (i, 0)),
        pl.BlockSpec((NB, DEGW), lambda i: (i, 0)),
        pl.BlockSpec((F2, F2), lambda i: (0, 0)),
        pl.BlockSpec((1, F2), lambda i: (0, 0)),
    ],
    out_specs=pl.BlockSpec((NB, F2), lambda i: (i, 0)),
    out_shape=jax.ShapeDtypeStruct((N, F2), jnp.float32),
)


def _tc_g_body(acc_ref, g2_ref, dinv_ref, batch_ref, b3_ref, wl1_ref, bl1_ref,
               wl2_ref, bl2_ref, out_ref, pool_ref, cnt_ref):
    i = pl.program_id(0)

    @pl.when(i == 0)
    def _init():
        pool_ref[...] = jnp.zeros_like(pool_ref)
        cnt_ref[...] = jnp.zeros_like(cnt_ref)

    dinv1 = dinv_ref[:, 0:1]
    h2 = dinv1 * (acc_ref[0] + acc_ref[1] + g2_ref[...]) + b3_ref[...]
    onehot = (batch_ref[...] == lax.broadcasted_iota(
        jnp.int32, (NB, G), 1)).astype(jnp.float32)
    dn = (((0,), (0,)), ((), ()))
    pool_ref[...] += lax.dot_general(
        onehot, h2, dn, preferred_element_type=jnp.float32)
    cnt_ref[...] += lax.dot_general(
        onehot, jnp.ones((NB, 1), jnp.float32), dn,
        preferred_element_type=jnp.float32)

    @pl.when(i == NBLK - 1)
    def _fin():
        pooled = pool_ref[...] / jnp.maximum(cnt_ref[...], 1.0)
        o1 = jnp.dot(pooled, wl1_ref[...],
                     preferred_element_type=jnp.float32) + bl1_ref[...]
        o2 = jnp.dot(o1, wl2_ref[...],
                     preferred_element_type=jnp.float32) + bl2_ref[...]
        m = jnp.max(o2, axis=-1, keepdims=True)
        ex = jnp.exp(o2 - m)
        out_ref[...] = ex / jnp.sum(ex, axis=-1, keepdims=True)


_tc_g = pl.pallas_call(
    _tc_g_body,
    grid=(NBLK,),
    in_specs=[
        pl.BlockSpec((NC, NB, F2), lambda i: (0, i, 0)),
        pl.BlockSpec((NB, F2), lambda i: (i, 0)),
        pl.BlockSpec((NB, DEGW), lambda i: (i, 0)),
        pl.BlockSpec((NB, 1), lambda i: (i, 0)),
        pl.BlockSpec((1, F2), lambda i: (0, 0)),
        pl.BlockSpec((F2, 32), lambda i: (0, 0)),
        pl.BlockSpec((1, 32), lambda i: (0, 0)),
        pl.BlockSpec((32, OUT), lambda i: (0, 0)),
        pl.BlockSpec((1, OUT), lambda i: (0, 0)),
    ],
    out_specs=pl.BlockSpec((G, OUT), lambda i: (0, 0)),
    out_shape=jax.ShapeDtypeStruct((G, OUT), jnp.float32),
    scratch_shapes=[
        pltpu.VMEM((G, F2), jnp.float32),
        pltpu.VMEM((G, 1), jnp.float32),
    ],
)


def kernel(x, edge_index, edge_attr, batch, W_rel, W_self, b_rgcn,
           W1, b1, W3, b3, Wl1, bl1, Wl2, bl2):
    src = edge_index[0]
    dst = edge_index[1]
    pad = E_PAD - E
    zpad = jnp.zeros((pad,), jnp.int32)
    gidx1 = jnp.concatenate([edge_attr * N + src, zpad]).reshape(NW, NCHUNK, CH)
    src_p = jnp.concatenate([src, zpad]).reshape(NW, NCHUNK, CH)
    didx = jnp.concatenate([dst, jnp.full((pad,), N, jnp.int32)]
                           ).reshape(NW, NCHUNK, CH)

    hrel, selfp = _tc_a(x, W_rel, W_self, b_rgcn.reshape(1, HID))
    acc0, degp = _edge_pass_rgcn(hrel.reshape(R * N, HID), gidx1, didx)
    g1, dinv = _tc_c(acc0, selfp, degp, W1)
    acc1, = _edge_pass_gcn(g1, src_p, didx)
    g2 = _tc_e(acc1, g1, dinv, W3, b1.reshape(1, F2))
    acc2, = _edge_pass_gcn(g2, src_p, didx)
    return _tc_g(acc2, g2, dinv, batch.reshape(N, 1), b3.reshape(1, F2),
                 Wl1, bl1.reshape(1, 32), Wl2, bl2.reshape(1, OUT))


# speedup vs baseline: 23.6996x; 23.6996x over previous
import functools

import jax
import jax.numpy as jnp
from jax import lax
from jax.experimental import pallas as pl
from jax.experimental.pallas import tpu as pltpu
from jax.experimental.pallas import tpu_sc as plsc

N, E, D, R, G = 10000, 320000, 128, 4, 64
HID, F2, OUT = 32, 64, 10

NC, NS = 2, 16
NW = NC * NS
CH = 128
EPT = E // NW
NCHUNK = -(-EPT // CH)
EPT_PAD = NCHUNK * CH
E_PAD = EPT_PAD * NW
NPAD = 10112
RPT = NPAD // NS
DEGW = 16
NB = 1000
NBLK = N // NB


def _edge_pass_body(with_deg, F, table_hbm, gidx_hbm, didx_hbm, *refs):
    if with_deg:
        acc_out, deg_out, gidx_v, didx_v, rows_v, acc_sh, ones_v, deg_sh = refs
    else:
        acc_out, gidx_v, didx_v, rows_v, acc_sh = refs
    c = lax.axis_index("c")
    s = lax.axis_index("s")
    wid = s * NC + c

    pltpu.sync_copy(gidx_hbm.at[wid], gidx_v)
    pltpu.sync_copy(didx_hbm.at[wid], didx_v)

    zv = jnp.zeros((16,), jnp.float32)

    def zrow(i, _):
        for k in range(F // 16):
            rows_v[i, pl.ds(k * 16, 16)] = zv
        return 0

    lax.fori_loop(0, CH, zrow, 0)
    base = s * RPT
    for off in range(0, RPT, CH):
        ln = min(CH, RPT - off)
        pltpu.sync_copy(rows_v.at[pl.ds(0, ln)], acc_sh.at[pl.ds(base + off, ln)])

    if with_deg:
        def zrow2(i, _):
            ones_v[i, pl.ds(0, 16)] = zv
            return 0

        lax.fori_loop(0, CH, zrow2, 0)
        for off in range(0, RPT, CH):
            ln = min(CH, RPT - off)
            pltpu.sync_copy(ones_v.at[pl.ds(0, ln)], deg_sh.at[pl.ds(base + off, ln)])
        ov = jnp.ones((16,), jnp.float32)

        def orow(i, _):
            ones_v[i, pl.ds(0, 16)] = ov
            return 0

        lax.fori_loop(0, CH, orow, 0)

    plsc.subcore_barrier()

    def chunk(j, _):
        pltpu.sync_copy(table_hbm.at[gidx_v.at[j]], rows_v)
        pltpu.sync_copy(rows_v, acc_sh.at[didx_v.at[j]], add=True)
        if with_deg:
            pltpu.sync_copy(ones_v, deg_sh.at[didx_v.at[j]], add=True)
        return 0

    lax.fori_loop(0, NCHUNK, chunk, 0)

    plsc.subcore_barrier()

    pltpu.sync_copy(acc_sh.at[pl.ds(base, RPT)], acc_out.at[c, pl.ds(base, RPT)])
    if with_deg:
        pltpu.sync_copy(deg_sh.at[pl.ds(base, RPT)], deg_out.at[c, pl.ds(base, RPT)])


def _make_edge_pass(F, with_deg):
    mesh = plsc.VectorSubcoreMesh(core_axis_name="c", subcore_axis_name="s")
    out_type = [jax.ShapeDtypeStruct((NC, NPAD, F), jnp.float32)]
    scratch = [
        pltpu.VMEM((NCHUNK, CH), jnp.int32),
        pltpu.VMEM((NCHUNK, CH), jnp.int32),
        pltpu.VMEM((CH, F), jnp.float32),
        pltpu.VMEM_SHARED((NPAD, F), jnp.float32),
    ]
    if with_deg:
        out_type.append(jax.ShapeDtypeStruct((NC, NPAD, DEGW), jnp.float32))
        scratch += [
            pltpu.VMEM((CH, DEGW), jnp.float32),
            pltpu.VMEM_SHARED((NPAD, DEGW), jnp.float32),
        ]
    return pl.kernel(
        functools.partial(_edge_pass_body, with_deg, F),
        out_type=out_type,
        mesh=mesh,
        scratch_types=scratch,
        compiler_params=pltpu.CompilerParams(use_tc_tiling_on_sc=False),
    )


_edge_pass_rgcn = _make_edge_pass(HID, True)
_edge_pass_gcn = _make_edge_pass(F2, False)


def _tc_a_body(x_ref, wrel_ref, wself_ref, brg_ref, hrel_ref, selfp_ref):
    xb = x_ref[...]
    for r in range(R):
        hrel_ref[r] = jnp.dot(xb, wrel_ref[r], preferred_element_type=jnp.float32)
    selfp_ref[...] = (
        jnp.dot(xb, wself_ref[...], preferred_element_type=jnp.float32)
        + brg_ref[...]
    )


_tc_a = pl.pallas_call(
    _tc_a_body,
    grid=(NBLK,),
    in_specs=[
        pl.BlockSpec((NB, D), lambda i: (i, 0)),
        pl.BlockSpec((R, D, HID), lambda i: (0, 0, 0)),
        pl.BlockSpec((D, HID), lambda i: (0, 0)),
        pl.BlockSpec((1, HID), lambda i: (0, 0)),
    ],
    out_specs=[
        pl.BlockSpec((R, NB, HID), lambda i: (0, i, 0)),
        pl.BlockSpec((NB, HID), lambda i: (i, 0)),
    ],
    out_shape=[
        jax.ShapeDtypeStruct((R, N, HID), jnp.float32),
        jax.ShapeDtypeStruct((N, HID), jnp.float32),
    ],
)


def _tc_c_body(acc_ref, selfp_ref, degp_ref, w1_ref, g1_ref, dinv_ref):
    h0 = jnp.maximum(acc_ref[0] + acc_ref[1] + selfp_ref[...], 0.0)
    deg = degp_ref[0] + degp_ref[1] + 1.0
    dinv = lax.rsqrt(deg)
    dinv_ref[...] = dinv
    g1_ref[...] = dinv[:, 0:1] * jnp.dot(
        h0, w1_ref[...], preferred_element_type=jnp.float32)


_tc_c = pl.pallas_call(
    _tc_c_body,
    grid=(NBLK,),
    in_specs=[
        pl.BlockSpec((NC, NB, HID), lambda i: (0, i, 0)),
        pl.BlockSpec((NB, HID), lambda i: (i, 0)),
        pl.BlockSpec((NC, NB, DEGW), lambda i: (0, i, 0)),
        pl.BlockSpec((HID, F2), lambda i: (0, 0)),
    ],
    out_specs=[
        pl.BlockSpec((NB, F2), lambda i: (i, 0)),
        pl.BlockSpec((NB, DEGW), lambda i: (i, 0)),
    ],
    out_shape=[
        jax.ShapeDtypeStruct((N, F2), jnp.float32),
        jax.ShapeDtypeStruct((N, DEGW), jnp.float32),
    ],
)


def _tc_e_body(acc_ref, g1_ref, dinv_ref, w3_ref, b1_ref, g2_ref):
    dinv1 = dinv_ref[:, 0:1]
    out1 = jnp.maximum(
        dinv1 * (acc_ref[0] + acc_ref[1] + g1_ref[...]) + b1_ref[...], 0.0)
    g2_ref[...] = dinv1 * jnp.dot(
        out1, w3_ref[...], preferred_element_type=jnp.float32)


_tc_e = pl.pallas_call(
    _tc_e_body,
    grid=(NBLK,),
    in_specs=[
        pl.BlockSpec((NC, NB, F2), lambda i: (0, i, 0)),
        pl.BlockSpec((NB, F2), lambda i: (i, 0)),
        pl.BlockSpec((NB, DEGW), lambda i: (i, 0)),
        pl.BlockSpec((F2, F2), lambda i: (0, 0)),
        pl.BlockSpec((1, F2), lambda i: (0, 0)),
    ],
    out_specs=pl.BlockSpec((NB, F2), lambda i: (i, 0)),
    out_shape=jax.ShapeDtypeStruct((N, F2), jnp.float32),
)


def _tc_g_body(acc_ref, g2_ref, dinv_ref, batch_ref, b3_ref, wl1_ref, bl1_ref,
               wl2_ref, bl2_ref, out_ref, pool_ref, cnt_ref):
    i = pl.program_id(0)

    @pl.when(i == 0)
    def _init():
        pool_ref[...] = jnp.zeros_like(pool_ref)
        cnt_ref[...] = jnp.zeros_like(cnt_ref)

    dinv1 = dinv_ref[:, 0:1]
    h2 = dinv1 * (acc_ref[0] + acc_ref[1] + g2_ref[...]) + b3_ref[...]
    onehot = (batch_ref[...] == lax.broadcasted_iota(
        jnp.int32, (NB, G), 1)).astype(jnp.float32)
    dn = (((0,), (0,)), ((), ()))
    pool_ref[...] += lax.dot_general(
        onehot, h2, dn, preferred_element_type=jnp.float32)
    cnt_ref[...] += lax.dot_general(
        onehot, jnp.ones((NB, 1), jnp.float32), dn,
        preferred_element_type=jnp.float32)

    @pl.when(i == NBLK - 1)
    def _fin():
        pooled = pool_ref[...] / jnp.maximum(cnt_ref[...], 1.0)
        o1 = jnp.dot(pooled, wl1_ref[...],
                     preferred_element_type=jnp.float32) + bl1_ref[...]
        o2 = jnp.dot(o1, wl2_ref[...],
                     preferred_element_type=jnp.float32) + bl2_ref[...]
        m = jnp.max(o2, axis=-1, keepdims=True)
        ex = jnp.exp(o2 - m)
        out_ref[...] = ex / jnp.sum(ex, axis=-1, keepdims=True)


_tc_g = pl.pallas_call(
    _tc_g_body,
    grid=(NBLK,),
    in_specs=[
        pl.BlockSpec((NC, NB, F2), lambda i: (0, i, 0)),
        pl.BlockSpec((NB, F2), lambda i: (i, 0)),
        pl.BlockSpec((NB, DEGW), lambda i: (i, 0)),
        pl.BlockSpec((NB, 1), lambda i: (i, 0)),
        pl.BlockSpec((1, F2), lambda i: (0, 0)),
        pl.BlockSpec((F2, 32), lambda i: (0, 0)),
        pl.BlockSpec((1, 32), lambda i: (0, 0)),
        pl.BlockSpec((32, OUT), lambda i: (0, 0)),
        pl.BlockSpec((1, OUT), lambda i: (0, 0)),
    ],
    out_specs=pl.BlockSpec((G, OUT), lambda i: (0, 0)),
    out_shape=jax.ShapeDtypeStruct((G, OUT), jnp.float32),
    scratch_shapes=[
        pltpu.VMEM((G, F2), jnp.float32),
        pltpu.VMEM((G, 1), jnp.float32),
    ],
)


def kernel(x, edge_index, edge_attr, batch, W_rel, W_self, b_rgcn,
           W1, b1, W3, b3, Wl1, bl1, Wl2, bl2):
    src = edge_index[0]
    dst = edge_index[1]
    pad = E_PAD - E
    zpad = jnp.zeros((pad,), jnp.int32)
    gidx1 = jnp.concatenate([edge_attr * N + src, zpad]).reshape(NW, NCHUNK, CH)
    src_p = jnp.concatenate([src, zpad]).reshape(NW, NCHUNK, CH)
    didx = jnp.concatenate([dst, jnp.full((pad,), N, jnp.int32)]
                           ).reshape(NW, NCHUNK, CH)

    hrel, selfp = _tc_a(x, W_rel, W_self, b_rgcn.reshape(1, HID))
    acc0, degp = _edge_pass_rgcn(hrel.reshape(R * N, HID), gidx1, didx)
    g1, dinv = _tc_c(acc0, selfp, degp, W1)
    acc1, = _edge_pass_gcn(g1, src_p, didx)
    g2 = _tc_e(acc1, g1, dinv, W3, b1.reshape(1, F2))
    acc2, = _edge_pass_gcn(g2, src_p, didx)
    return _tc_g(acc2, g2, dinv, batch.reshape(N, 1), b3.reshape(1, F2),
                 Wl1, bl1.reshape(1, 32), Wl2, bl2.reshape(1, OUT))
